# baseline (speedup 1.0000x reference)
import functools
import math

import numpy as np
import jax
import jax.numpy as jnp
from jax.experimental import pallas as pl
from jax.experimental.pallas import tpu as pltpu

_N_CHROMA = 12
_HIDDEN = 512
_LATENT = 256
_LAYERS = 4
_HEADS = 8
_DEPTH = 8
_WINDOW = 16
_DH = _HIDDEN // _HEADS

_QB = 128
_PAIR = 4
_XB = 1024
_NEG = -1e30


def _cantor_coordinates(T):
    coords = np.zeros((T,), dtype=np.float64)
    for pos in range(T):
        x = pos / max(1, T - 1)
        x = max(1e-06, min(x, 1.0 - 1e-06))
        val = 0.0
        factor = 0.5
        for _ in range(_DEPTH):
            x *= 3.0
            digit = int(x)
            x -= digit
            if digit == 2:
                val += factor
            factor *= 0.5
        coords[pos] = val
    return coords


@functools.lru_cache(maxsize=None)
def _routing(T):
    coords = _cantor_coordinates(T)
    w = min(_WINDOW, T)
    routes = np.zeros((T, _WINDOW), dtype=np.int64)
    for i in range(T):
        d = np.abs(coords - coords[i])
        idx = np.argsort(d, kind='stable')[:w]
        routes[i, :w] = idx
    perm = np.argsort(coords, kind='stable')
    rank = np.empty((T,), dtype=np.int64)
    rank[perm] = np.arange(T)
    r_ranks = rank[routes]

    nblk = T // _QB
    offs = np.zeros((nblk,), dtype=np.int32)
    width = 0
    for q in range(nblk):
        toks = perm[q * _QB:(q + 1) * _QB]
        lo = int(r_ranks[toks].min())
        hi = int(r_ranks[toks].max())
        lo = (lo // 8) * 8
        offs[q] = lo
        width = max(width, hi - lo + 1)
    KW = min(T, max(128, ((width + 127) // 128) * 128))
    offs = np.minimum(offs, T - KW).astype(np.int32)

    mask = np.full((nblk, _QB, KW), _NEG, dtype=np.float32)
    for q in range(nblk):
        toks = perm[q * _QB:(q + 1) * _QB]
        for r in range(_QB):
            cols = r_ranks[toks[r]] - offs[q]
            mask[q, r, cols] = 0.0
    return perm, rank, offs, mask, KW


@functools.lru_cache(maxsize=None)
def _circular_enc():
    enc = np.zeros((_N_CHROMA, _HIDDEN), dtype=np.float32)
    for i in range(_N_CHROMA):
        for j in range(_HIDDEN // 2):
            freq = (j + 1) / (_HIDDEN / 2)
            angle = 2 * math.pi * i * freq / _N_CHROMA
            enc[i, 2 * j] = math.cos(angle)
            enc[i, 2 * j + 1] = math.sin(angle)
    return enc


def _ln(y, g, b):
    m = jnp.mean(y, axis=-1, keepdims=True)
    d = y - m
    v = jnp.mean(d * d, axis=-1, keepdims=True)
    return d * jax.lax.rsqrt(v + 1e-05) * g + b


def _dot(a, b, trans_b=False):
    dn = (((1,), (1 if trans_b else 0,)), ((), ()))
    return jax.lax.dot_general(a, b, dn, preferred_element_type=jnp.float32)


def _gelu(x):
    return 0.5 * x * (1.0 + jax.lax.erf(x * (1.0 / math.sqrt(2.0))))


def _write_qkv(acc, t_ref, qkv_ref):
    scale = 1.0 / (math.sqrt(_DH) * jnp.abs(t_ref[0]))
    qkv_ref[0, :, 0:_HIDDEN] = (acc[:, 0:_HIDDEN] * scale).astype(jnp.bfloat16)
    qkv_ref[0, :, _HIDDEN:] = acc[:, _HIDDEN:].astype(jnp.bfloat16)


def _embed_qkv_body(t_ref, c_ref, we_ref, enc_ref, be_ref, wq_ref, bq_ref,
                    x_ref, qkv_ref):
    w = we_ref[...] + enc_ref[...]
    x = _dot(c_ref[0], w) + be_ref[...]
    x_ref[0] = x
    acc = _dot(x.astype(jnp.bfloat16), wq_ref[...]) + bq_ref[...]
    _write_qkv(acc, t_ref, qkv_ref)


def _attn_body(offs_ref, qkv_ref, m_ref, a_ref, *, KW, shift):
    qq = pl.program_id(1)
    for j in range(_PAIR):
        off = pl.multiple_of(offs_ref[qq * _PAIR + j], 8)
        row0 = (qq * _PAIR + j) * _QB
        Qb = qkv_ref[0, pl.ds(row0, _QB), 0:_HIDDEN]
        K = qkv_ref[0, pl.ds(off, KW), _HIDDEN:2 * _HIDDEN]
        V = qkv_ref[0, pl.ds(off, KW), 2 * _HIDDEN:3 * _HIDDEN]
        mask = m_ref[0, j * _QB:(j + 1) * _QB, :]

        outs = []
        for h in range(_HEADS):
            sl = slice(h * _DH, (h + 1) * _DH)
            s = _dot(Qb[:, sl], K[:, sl], trans_b=True) + mask
            if shift:
                s = s - jnp.max(s, axis=-1, keepdims=True)
            e = jnp.exp(s)
            o = _dot(e.astype(jnp.bfloat16), V[:, sl])
            outs.append(o / jnp.sum(e, axis=-1, keepdims=True))
        a_ref[0, j * _QB:(j + 1) * _QB, :] = (
            jnp.concatenate(outs, axis=-1).astype(jnp.bfloat16))


def _post_attn(a_ref, x_ref, wo_ref, bo_ref, g_ref, be_ref,
               w1_ref, b1_ref, w2_ref, b2_ref):
    y = _dot(a_ref[0], wo_ref[...]) + bo_ref[...] + x_ref[0]
    y = _ln(y, g_ref[...], be_ref[...])
    h = _gelu(_dot(y.astype(jnp.bfloat16), w1_ref[...]) + b1_ref[...])
    z = _dot(h.astype(jnp.bfloat16), w2_ref[...]) + b2_ref[...] + y
    return _ln(z, g_ref[...], be_ref[...])


def _ffn_qkv_body(t_ref, a_ref, x_ref, wo_ref, bo_ref, g_ref, be_ref,
                  w1_ref, b1_ref, w2_ref, b2_ref, wq_ref, bq_ref,
                  x_out_ref, qkv_ref):
    z = _post_attn(a_ref, x_ref, wo_ref, bo_ref, g_ref, be_ref,
                   w1_ref, b1_ref, w2_ref, b2_ref)
    x_out_ref[0] = z
    acc = _dot(z.astype(jnp.bfloat16), wq_ref[...]) + bq_ref[...]
    _write_qkv(acc, t_ref, qkv_ref)


def _ffn_final_body(a_ref, x_ref, wo_ref, bo_ref, g_ref, be_ref,
                    w1_ref, b1_ref, w2_ref, b2_ref,
                    wm_ref, bm_ref, mu_ref, lv_ref):
    z = _post_attn(a_ref, x_ref, wo_ref, bo_ref, g_ref, be_ref,
                   w1_ref, b1_ref, w2_ref, b2_ref)
    acc = _dot(z.astype(jnp.bfloat16), wm_ref[...]) + bm_ref[...]
    mu_ref[0] = acc[:, 0:_LATENT]
    lv_ref[0] = acc[:, _LATENT:]


def _vspec(block, index_map):
    return pl.BlockSpec(block, index_map)


def _const_spec(shape):
    return pl.BlockSpec(shape, lambda *_: (0,) * len(shape))


def kernel(chroma, params):
    B, T, C = chroma.shape
    assert C == _N_CHROMA and T % _XB == 0 and T % _QB == 0
    perm, rank, offs_np, mask_np, KW = _routing(T)
    nblk = T // _QB
    f32, bf16 = jnp.float32, jnp.bfloat16
    H, FF, L3 = _HIDDEN, 4 * _HIDDEN, 3 * _HIDDEN
    smem = pl.BlockSpec(memory_space=pltpu.SMEM)

    def wqkv_of(l):
        return (jnp.concatenate([params['Wq'][l], params['Wk'][l],
                                 params['Wv'][l]], axis=1).astype(bf16),
                jnp.concatenate([params['bq'][l], params['bk'][l],
                                 params['bv'][l]]).reshape(1, L3),
                params['temp'][l].reshape(1))

    xp = jnp.take(chroma, jnp.asarray(perm), axis=1)

    enc = jnp.asarray(_circular_enc())
    wq0, bq0, t0 = wqkv_of(0)
    x, qkv = pl.pallas_call(
        _embed_qkv_body,
        grid=(B,),
        in_specs=[smem,
                  _vspec((1, T, C), lambda b: (b, 0, 0)),
                  _const_spec((C, H)),
                  _const_spec((C, H)),
                  _const_spec((1, H)),
                  _const_spec((H, L3)),
                  _const_spec((1, L3))],
        out_specs=(_vspec((1, T, H), lambda b: (b, 0, 0)),
                   _vspec((1, T, L3), lambda b: (b, 0, 0))),
        out_shape=(jax.ShapeDtypeStruct((B, T, H), f32),
                   jax.ShapeDtypeStruct((B, T, L3), bf16)),
    )(t0, xp, params['emb_W'], enc, params['emb_b'].reshape(1, H), wq0, bq0)

    offs = jnp.asarray(offs_np)
    mask = jnp.asarray(mask_np.reshape(nblk // _PAIR, _PAIR * _QB, KW))

    def attn_call_for(shift):
        return pl.pallas_call(
            functools.partial(_attn_body, KW=KW, shift=shift),
            grid=(B, nblk // _PAIR),
            in_specs=[smem,
                      _vspec((1, T, L3), lambda b, q: (b, 0, 0)),
                      _vspec((1, _PAIR * _QB, KW), lambda b, q: (q, 0, 0))],
            out_specs=_vspec((1, _PAIR * _QB, H), lambda b, q: (b, q, 0)),
            out_shape=jax.ShapeDtypeStruct((B, T, H), bf16),
        )
    attn_calls = [attn_call_for(True), attn_call_for(False)]

    ffn_qkv_call = pl.pallas_call(
        _ffn_qkv_body,
        grid=(B, T // _XB),
        in_specs=[smem,
                  _vspec((1, _XB, H), lambda b, i: (b, i, 0)),
                  _vspec((1, _XB, H), lambda b, i: (b, i, 0)),
                  _const_spec((H, H)),
                  _const_spec((1, H)),
                  _const_spec((1, H)),
                  _const_spec((1, H)),
                  _const_spec((H, FF)),
                  _const_spec((1, FF)),
                  _const_spec((FF, H)),
                  _const_spec((1, H)),
                  _const_spec((H, L3)),
                  _const_spec((1, L3))],
        out_specs=(_vspec((1, _XB, H), lambda b, i: (b, i, 0)),
                   _vspec((1, _XB, L3), lambda b, i: (b, i, 0))),
        out_shape=(jax.ShapeDtypeStruct((B, T, H), f32),
                   jax.ShapeDtypeStruct((B, T, L3), bf16)),
    )

    wml = jnp.concatenate([params['Wmu'], params['Wlv']], axis=1).astype(bf16)
    bml = jnp.concatenate([params['bmu'], params['blv']]).reshape(1, 2 * _LATENT)

    for l in range(_LAYERS):
        ln_g = params['ln_g'][l].reshape(1, H)
        ln_b = params['ln_b'][l].reshape(1, H)
        attn = attn_calls[min(l, 1)](offs, qkv, mask)
        wo = params['Wo'][l].astype(bf16)
        bo = params['bo'][l].reshape(1, H)
        w1 = params['W1'][l].astype(bf16)
        b1 = params['b1'][l].reshape(1, FF)
        w2 = params['W2'][l].astype(bf16)
        b2 = params['b2'][l].reshape(1, H)
        if l < _LAYERS - 1:
            wqn, bqn, tn = wqkv_of(l + 1)
            x, qkv = ffn_qkv_call(tn, attn, x, wo, bo, ln_g, ln_b,
                                  w1, b1, w2, b2, wqn, bqn)
        else:
            mu_p, lv_p = pl.pallas_call(
                _ffn_final_body,
                grid=(B, T // _XB),
                in_specs=[_vspec((1, _XB, H), lambda b, i: (b, i, 0)),
                          _vspec((1, _XB, H), lambda b, i: (b, i, 0)),
                          _const_spec((H, H)),
                          _const_spec((1, H)),
                          _const_spec((1, H)),
                          _const_spec((1, H)),
                          _const_spec((H, FF)),
                          _const_spec((1, FF)),
                          _const_spec((FF, H)),
                          _const_spec((1, H)),
                          _const_spec((H, 2 * _LATENT)),
                          _const_spec((1, 2 * _LATENT))],
                out_specs=(_vspec((1, _XB, _LATENT), lambda b, i: (b, i, 0)),
                           _vspec((1, _XB, _LATENT), lambda b, i: (b, i, 0))),
                out_shape=(jax.ShapeDtypeStruct((B, T, _LATENT), f32),
                           jax.ShapeDtypeStruct((B, T, _LATENT), f32)),
            )(attn, x, wo, bo, ln_g, ln_b, w1, b1, w2, b2, wml, bml)

    r = jnp.asarray(rank)
    return (jnp.take(mu_p, r, axis=1), jnp.take(lv_p, r, axis=1))


# speedup vs baseline: 1.1322x; 1.1322x over previous
import functools
import math

import numpy as np
import jax
import jax.numpy as jnp
from jax.experimental import pallas as pl
from jax.experimental.pallas import tpu as pltpu

_N_CHROMA = 12
_HIDDEN = 512
_LATENT = 256
_LAYERS = 4
_HEADS = 8
_DEPTH = 8
_WINDOW = 16
_DH = _HIDDEN // _HEADS

_QB = 128
_PAIR = 2
_XB = 1024
_NEG = -1e30


def _cantor_coordinates(T):
    coords = np.zeros((T,), dtype=np.float64)
    for pos in range(T):
        x = pos / max(1, T - 1)
        x = max(1e-06, min(x, 1.0 - 1e-06))
        val = 0.0
        factor = 0.5
        for _ in range(_DEPTH):
            x *= 3.0
            digit = int(x)
            x -= digit
            if digit == 2:
                val += factor
            factor *= 0.5
        coords[pos] = val
    return coords


@functools.lru_cache(maxsize=None)
def _routing(T):
    coords = _cantor_coordinates(T)
    w = min(_WINDOW, T)
    routes = np.zeros((T, _WINDOW), dtype=np.int64)
    for i in range(T):
        d = np.abs(coords - coords[i])
        idx = np.argsort(d, kind='stable')[:w]
        routes[i, :w] = idx
    perm = np.argsort(coords, kind='stable')
    rank = np.empty((T,), dtype=np.int64)
    rank[perm] = np.arange(T)
    r_ranks = rank[routes]

    nblk = T // _QB
    offs = np.zeros((nblk,), dtype=np.int32)
    width = 0
    for q in range(nblk):
        toks = perm[q * _QB:(q + 1) * _QB]
        lo = int(r_ranks[toks].min())
        hi = int(r_ranks[toks].max())
        lo = (lo // 8) * 8
        offs[q] = lo
        width = max(width, hi - lo + 1)
    KW = min(T, max(128, ((width + 127) // 128) * 128))
    offs = np.minimum(offs, T - KW).astype(np.int32)

    mask = np.full((nblk, _QB, KW), _NEG, dtype=np.float32)
    for q in range(nblk):
        toks = perm[q * _QB:(q + 1) * _QB]
        for r in range(_QB):
            cols = r_ranks[toks[r]] - offs[q]
            mask[q, r, cols] = 0.0
    return perm, rank, offs, mask, KW


@functools.lru_cache(maxsize=None)
def _circular_enc():
    enc = np.zeros((_N_CHROMA, _HIDDEN), dtype=np.float32)
    for i in range(_N_CHROMA):
        for j in range(_HIDDEN // 2):
            freq = (j + 1) / (_HIDDEN / 2)
            angle = 2 * math.pi * i * freq / _N_CHROMA
            enc[i, 2 * j] = math.cos(angle)
            enc[i, 2 * j + 1] = math.sin(angle)
    return enc


def _ln(y, g, b):
    m = jnp.mean(y, axis=-1, keepdims=True)
    d = y - m
    v = jnp.mean(d * d, axis=-1, keepdims=True)
    return d * jax.lax.rsqrt(v + 1e-05) * g + b


def _dot(a, b, trans_b=False):
    dn = (((1,), (1 if trans_b else 0,)), ((), ()))
    return jax.lax.dot_general(a, b, dn, preferred_element_type=jnp.float32)


def _gelu(x):
    return 0.5 * x * (1.0 + jax.lax.erf(x * (1.0 / math.sqrt(2.0))))


def _bf(ref):
    return ref[0].astype(jnp.bfloat16)


def _write_qkv(z16, t_ref, l, wq_ref, wk_ref, wv_ref,
               bq_ref, bk_ref, bv_ref, qkv_ref):
    scale = 1.0 / (math.sqrt(_DH) * jnp.abs(t_ref[l]))
    qv = (_dot(z16, _bf(wq_ref)) + bq_ref[0]) * scale
    qkv_ref[0, :, 0:_HIDDEN] = qv.astype(jnp.bfloat16)
    kv = _dot(z16, _bf(wk_ref)) + bk_ref[0]
    qkv_ref[0, :, _HIDDEN:2 * _HIDDEN] = kv.astype(jnp.bfloat16)
    vv = _dot(z16, _bf(wv_ref)) + bv_ref[0]
    qkv_ref[0, :, 2 * _HIDDEN:3 * _HIDDEN] = vv.astype(jnp.bfloat16)


def _embed_qkv_body(t_ref, c_ref, we_ref, enc_ref, be_ref,
                    wq_ref, wk_ref, wv_ref, bq_ref, bk_ref, bv_ref,
                    x_ref, qkv_ref):
    w = we_ref[...] + enc_ref[...]
    x = _dot(c_ref[0], w) + be_ref[...]
    x_ref[0] = x
    _write_qkv(x.astype(jnp.bfloat16), t_ref, 0,
               wq_ref, wk_ref, wv_ref, bq_ref, bk_ref, bv_ref, qkv_ref)


def _attn_body(offs_ref, qkv_ref, m_ref, a_ref, *, KW, shift):
    qq = pl.program_id(1)
    for j in range(_PAIR):
        off = pl.multiple_of(offs_ref[qq * _PAIR + j], 8)
        row0 = (qq * _PAIR + j) * _QB
        Qb = qkv_ref[0, pl.ds(row0, _QB), 0:_HIDDEN]
        K = qkv_ref[0, pl.ds(off, KW), _HIDDEN:2 * _HIDDEN]
        V = qkv_ref[0, pl.ds(off, KW), 2 * _HIDDEN:3 * _HIDDEN]
        mask = m_ref[0, j * _QB:(j + 1) * _QB, :]

        outs = []
        for h in range(_HEADS):
            sl = slice(h * _DH, (h + 1) * _DH)
            s = _dot(Qb[:, sl], K[:, sl], trans_b=True) + mask
            if shift:
                s = s - jnp.max(s, axis=-1, keepdims=True)
            e = jnp.exp(s)
            o = _dot(e.astype(jnp.bfloat16), V[:, sl])
            outs.append(o / jnp.sum(e, axis=-1, keepdims=True))
        a_ref[0, j * _QB:(j + 1) * _QB, :] = (
            jnp.concatenate(outs, axis=-1).astype(jnp.bfloat16))


def _post_attn(a_ref, x_ref, wo_ref, bo_ref, g_ref, be_ref,
               w1_ref, b1_ref, w2_ref, b2_ref):
    y = _dot(a_ref[0], _bf(wo_ref)) + bo_ref[0] + x_ref[0]
    y = _ln(y, g_ref[0], be_ref[0])
    h = _gelu(_dot(y.astype(jnp.bfloat16), _bf(w1_ref)) + b1_ref[0])
    z = _dot(h.astype(jnp.bfloat16), _bf(w2_ref)) + b2_ref[0] + y
    return _ln(z, g_ref[0], be_ref[0])


def _ffn_qkv_body(t_ref, a_ref, x_ref, wo_ref, bo_ref, g_ref, be_ref,
                  w1_ref, b1_ref, w2_ref, b2_ref,
                  wq_ref, wk_ref, wv_ref, bq_ref, bk_ref, bv_ref,
                  x_out_ref, qkv_ref, *, l):
    z = _post_attn(a_ref, x_ref, wo_ref, bo_ref, g_ref, be_ref,
                   w1_ref, b1_ref, w2_ref, b2_ref)
    x_out_ref[0] = z
    _write_qkv(z.astype(jnp.bfloat16), t_ref, l,
               wq_ref, wk_ref, wv_ref, bq_ref, bk_ref, bv_ref, qkv_ref)


def _ffn_final_body(a_ref, x_ref, wo_ref, bo_ref, g_ref, be_ref,
                    w1_ref, b1_ref, w2_ref, b2_ref,
                    wmu_ref, bmu_ref, wlv_ref, blv_ref, mu_ref, lv_ref):
    z = _post_attn(a_ref, x_ref, wo_ref, bo_ref, g_ref, be_ref,
                   w1_ref, b1_ref, w2_ref, b2_ref)
    z16 = z.astype(jnp.bfloat16)
    mu_ref[0] = _dot(z16, wmu_ref[...].astype(jnp.bfloat16)) + bmu_ref[...]
    lv_ref[0] = _dot(z16, wlv_ref[...].astype(jnp.bfloat16)) + blv_ref[...]


def _vspec(block, index_map):
    return pl.BlockSpec(block, index_map)


def _const_spec(shape):
    return pl.BlockSpec(shape, lambda *_: (0,) * len(shape))


def _layer_spec(shape, l):
    return pl.BlockSpec(shape, lambda *_, _l=l: (_l,) + (0,) * (len(shape) - 1))


def kernel(chroma, params):
    B, T, C = chroma.shape
    assert C == _N_CHROMA and T % _XB == 0 and T % (_QB * _PAIR) == 0
    perm, rank, offs_np, mask_np, KW = _routing(T)
    nblk = T // _QB
    f32, bf16 = jnp.float32, jnp.bfloat16
    H, FF = _HIDDEN, 4 * _HIDDEN
    smem = pl.BlockSpec(memory_space=pltpu.SMEM)

    xp = jnp.take(chroma, jnp.asarray(perm), axis=1)
    enc = jnp.asarray(_circular_enc())

    def qkv_weight_args(l):
        L = _LAYERS
        specs = [_layer_spec((1, H, H), l), _layer_spec((1, H, H), l),
                 _layer_spec((1, H, H), l), _layer_spec((1, 1, H), l),
                 _layer_spec((1, 1, H), l), _layer_spec((1, 1, H), l)]
        vals = (params['Wq'], params['Wk'], params['Wv'],
                params['bq'].reshape(L, 1, H), params['bk'].reshape(L, 1, H),
                params['bv'].reshape(L, 1, H))
        return specs, vals

    qspecs0, qvals0 = qkv_weight_args(0)
    x, qkv = pl.pallas_call(
        _embed_qkv_body,
        grid=(B,),
        in_specs=[smem,
                  _vspec((1, T, C), lambda b: (b, 0, 0)),
                  _const_spec((C, H)),
                  _const_spec((C, H)),
                  _const_spec((1, H))] + qspecs0,
        out_specs=(_vspec((1, T, H), lambda b: (b, 0, 0)),
                   _vspec((1, T, 3 * H), lambda b: (b, 0, 0))),
        out_shape=(jax.ShapeDtypeStruct((B, T, H), f32),
                   jax.ShapeDtypeStruct((B, T, 3 * H), bf16)),
    )(params['temp'], xp, params['emb_W'], enc,
      params['emb_b'].reshape(1, H), *qvals0)

    offs = jnp.asarray(offs_np)
    mask = jnp.asarray(mask_np.reshape(nblk // _PAIR, _PAIR * _QB, KW))

    def attn_call_for(shift):
        return pl.pallas_call(
            functools.partial(_attn_body, KW=KW, shift=shift),
            grid=(B, nblk // _PAIR),
            in_specs=[smem,
                      _vspec((1, T, 3 * H), lambda b, q: (b, 0, 0)),
                      _vspec((1, _PAIR * _QB, KW), lambda b, q: (q, 0, 0))],
            out_specs=_vspec((1, _PAIR * _QB, H), lambda b, q: (b, q, 0)),
            out_shape=jax.ShapeDtypeStruct((B, T, H), bf16),
        )
    attn_calls = [attn_call_for(True), attn_call_for(False)]

    def layer_common(l):
        specs = [_vspec((1, _XB, H), lambda b, i: (b, i, 0)),
                 _vspec((1, _XB, H), lambda b, i: (b, i, 0)),
                 _layer_spec((1, H, H), l), _layer_spec((1, 1, H), l),
                 _layer_spec((1, 1, H), l), _layer_spec((1, 1, H), l),
                 _layer_spec((1, H, FF), l), _layer_spec((1, 1, FF), l),
                 _layer_spec((1, FF, H), l), _layer_spec((1, 1, H), l)]
        L = _LAYERS
        vals = (params['Wo'], params['bo'].reshape(L, 1, H),
                params['ln_g'].reshape(L, 1, H), params['ln_b'].reshape(L, 1, H),
                params['W1'], params['b1'].reshape(L, 1, FF),
                params['W2'], params['b2'].reshape(L, 1, H))
        return specs, vals

    for l in range(_LAYERS):
        attn = attn_calls[min(l, 1)](offs, qkv, mask)
        cspecs, cvals = layer_common(l)
        if l < _LAYERS - 1:
            qspecs, qvals = qkv_weight_args(l + 1)
            x, qkv = pl.pallas_call(
                functools.partial(_ffn_qkv_body, l=l + 1),
                grid=(B, T // _XB),
                in_specs=[smem] + cspecs + qspecs,
                out_specs=(_vspec((1, _XB, H), lambda b, i: (b, i, 0)),
                           _vspec((1, _XB, 3 * H), lambda b, i: (b, i, 0))),
                out_shape=(jax.ShapeDtypeStruct((B, T, H), f32),
                           jax.ShapeDtypeStruct((B, T, 3 * H), bf16)),
            )(params['temp'], attn, x, *cvals, *qvals)
        else:
            mu_p, lv_p = pl.pallas_call(
                _ffn_final_body,
                grid=(B, T // _XB),
                in_specs=cspecs + [_const_spec((H, _LATENT)),
                                   _const_spec((1, _LATENT)),
                                   _const_spec((H, _LATENT)),
                                   _const_spec((1, _LATENT))],
                out_specs=(_vspec((1, _XB, _LATENT), lambda b, i: (b, i, 0)),
                           _vspec((1, _XB, _LATENT), lambda b, i: (b, i, 0))),
                out_shape=(jax.ShapeDtypeStruct((B, T, _LATENT), f32),
                           jax.ShapeDtypeStruct((B, T, _LATENT), f32)),
            )(attn, x, *cvals, params['Wmu'],
              params['bmu'].reshape(1, _LATENT), params['Wlv'],
              params['blv'].reshape(1, _LATENT))

    r = jnp.asarray(rank)
    return (jnp.take(mu_p, r, axis=1), jnp.take(lv_p, r, axis=1))
